# baseline (speedup 1.0000x reference)
import functools

import jax
import jax.numpy as jnp
from jax.experimental import pallas as pl

_SCALING = 32.0 / 16.0


def _fused_kernel(x_ref, wb_ref, b_ref, rb_ref, ext_ref, bcat_ref, o_ref, *,
                  n_route: int, n_exp: int, rank: int):
    f32 = jnp.float32
    xm = x_ref[...]
    base = jax.lax.dot_general(xm, wb_ref[...], (((1,), (1,)), ((), ())),
                               preferred_element_type=f32)
    ext = jax.lax.dot_general(xm, ext_ref[...], (((1,), (1,)), ((), ())),
                              preferred_element_type=f32)
    logits = ext[:, :n_route] + rb_ref[...]
    ax = ext[:, 128:128 + n_exp * rank]

    bm = logits.shape[0]
    col = jax.lax.broadcasted_iota(jnp.int32, (bm, n_route), 1)
    rank_ct = jnp.zeros((bm, n_route), dtype=jnp.int32)
    for j in range(n_route):
        cj = logits[:, j:j + 1]
        beats = (cj > logits) | ((cj == logits) & (j < col))
        rank_ct = rank_ct + beats.astype(jnp.int32)
    sel = (rank_ct < n_exp).astype(f32)

    mx = jnp.max(logits, axis=1, keepdims=True)
    ex = jnp.exp(logits - mx)
    probs = ex / jnp.sum(ex, axis=1, keepdims=True)
    wsel = probs * sel

    w0 = jnp.sum(wsel[:, :n_exp], axis=1, keepdims=True)
    parts = [jnp.broadcast_to(w0, (bm, rank))]
    for e in range(1, n_exp):
        parts.append(jnp.broadcast_to(wsel[:, n_exp - 1 + e:n_exp + e],
                                      (bm, rank)))
    wfull = jnp.concatenate(parts, axis=1)

    axw = (ax * wfull).astype(jnp.bfloat16)
    lora = jax.lax.dot_general(axw, bcat_ref[...], (((1,), (0,)), ((), ())),
                               preferred_element_type=f32)
    o_ref[...] = base + b_ref[...] + _SCALING * lora


def kernel(x, W_base, b_base, router_W, router_b, A, Bm, expert_map):
    B, S, D = x.shape
    OUT = W_base.shape[0]
    E, R, _ = A.shape
    n_route = router_W.shape[0]
    M = B * S
    bf16 = jnp.bfloat16

    xf = x.reshape(M, D).astype(bf16)
    ext = jnp.zeros((128 + E * R, D), dtype=jnp.float32)
    ext = ext.at[:n_route].set(router_W)
    ext = ext.at[128:].set(A.reshape(E * R, D))
    ext = ext.astype(bf16)
    bcat = Bm.transpose(0, 2, 1).reshape(E * R, OUT).astype(bf16)
    wb = W_base.astype(bf16)
    b2 = b_base.reshape(1, OUT)
    rb2 = router_b.reshape(1, n_route)

    bm = 512
    while M % bm != 0:
        bm //= 2
    grid = (M // bm,)

    out = pl.pallas_call(
        functools.partial(_fused_kernel, n_route=n_route, n_exp=E, rank=R),
        grid=grid,
        in_specs=[
            pl.BlockSpec((bm, D), lambda i: (i, 0)),
            pl.BlockSpec((OUT, D), lambda i: (0, 0)),
            pl.BlockSpec((1, OUT), lambda i: (0, 0)),
            pl.BlockSpec((1, n_route), lambda i: (0, 0)),
            pl.BlockSpec((128 + E * R, D), lambda i: (0, 0)),
            pl.BlockSpec((E * R, OUT), lambda i: (0, 0)),
        ],
        out_specs=pl.BlockSpec((bm, OUT), lambda i: (i, 0)),
        out_shape=jax.ShapeDtypeStruct((M, OUT), jnp.float32),
    )(xf, wb, b2, rb2, ext, bcat)
    return out.reshape(B, S, OUT)


# speedup vs baseline: 1.9485x; 1.9485x over previous
import functools

import jax
import jax.numpy as jnp
from jax.experimental import pallas as pl

_SCALING = 32.0 / 16.0


def _fused_kernel(x_ref, wb_ref, b_ref, rw_ref, rb_ref, emap_ref, acat_ref,
                  bcat_ref, o_ref, *, n_route: int, n_exp: int, rank: int):
    f32 = jnp.float32
    xm = x_ref[...]
    base = jax.lax.dot_general(xm, wb_ref[...], (((1,), (1,)), ((), ())),
                               preferred_element_type=f32)
    ax = jax.lax.dot_general(xm, acat_ref[...], (((1,), (1,)), ((), ())),
                             preferred_element_type=f32)
    logitsT = jax.lax.dot_general(rw_ref[...], xm, (((1,), (1,)), ((), ())),
                                  preferred_element_type=f32) + rb_ref[...]

    bm = xm.shape[0]
    row = jax.lax.broadcasted_iota(jnp.int32, (n_route, bm), 0)
    rank_ct = jnp.zeros((n_route, bm), dtype=jnp.int32)
    for j in range(n_route):
        cj = logitsT[j:j + 1, :]
        beats = (cj > logitsT) | ((cj == logitsT) & (j < row))
        rank_ct = rank_ct + beats.astype(jnp.int32)
    sel = (rank_ct < n_exp).astype(f32)

    mx = jnp.max(logitsT, axis=0, keepdims=True)
    ex = jnp.exp(logitsT - mx)
    probs = ex / jnp.sum(ex, axis=0, keepdims=True)
    wselT = probs * sel

    lane_exp = jax.lax.broadcasted_iota(jnp.int32, (n_route, n_exp * rank),
                                        1) // rank
    smap = (emap_ref[...] == lane_exp).astype(f32)
    wfull = jax.lax.dot_general(wselT, smap, (((0,), (0,)), ((), ())),
                                preferred_element_type=f32)

    lora = jax.lax.dot_general(ax * wfull, bcat_ref[...],
                               (((1,), (0,)), ((), ())),
                               preferred_element_type=f32)
    o_ref[...] = base + b_ref[...] + _SCALING * lora


def kernel(x, W_base, b_base, router_W, router_b, A, Bm, expert_map):
    B, S, D = x.shape
    OUT = W_base.shape[0]
    E, R, _ = A.shape
    n_route = router_W.shape[0]
    M = B * S

    xf = x.reshape(M, D)
    acat = A.reshape(E * R, D)
    bcat = Bm.transpose(0, 2, 1).reshape(E * R, OUT)
    b2 = b_base.reshape(1, OUT)
    rb2 = router_b.reshape(n_route, 1)
    emap2 = expert_map.reshape(n_route, 1)

    bm = 512
    while M % bm != 0:
        bm //= 2
    grid = (M // bm,)

    out = pl.pallas_call(
        functools.partial(_fused_kernel, n_route=n_route, n_exp=E, rank=R),
        grid=grid,
        in_specs=[
            pl.BlockSpec((bm, D), lambda i: (i, 0)),
            pl.BlockSpec((OUT, D), lambda i: (0, 0)),
            pl.BlockSpec((1, OUT), lambda i: (0, 0)),
            pl.BlockSpec((n_route, D), lambda i: (0, 0)),
            pl.BlockSpec((n_route, 1), lambda i: (0, 0)),
            pl.BlockSpec((n_route, 1), lambda i: (0, 0)),
            pl.BlockSpec((E * R, D), lambda i: (0, 0)),
            pl.BlockSpec((E * R, OUT), lambda i: (0, 0)),
        ],
        out_specs=pl.BlockSpec((bm, OUT), lambda i: (i, 0)),
        out_shape=jax.ShapeDtypeStruct((M, OUT), jnp.float32),
    )(xf, W_base, b2, router_W, rb2, emap2, acat, bcat)
    return out.reshape(B, S, OUT)
